# baseline (speedup 1.0000x reference)
import jax
import jax.numpy as jnp
from jax import lax
from jax.experimental import pallas as pl
from jax.experimental.pallas import tpu as pltpu
from jax.experimental.pallas import tpu_sc as plsc

_N = 5_000_000
_B = 1_000_000

_NC = 2
_NS = 16
_NW = _NC * _NS
_CHUNK = 31_360
_B_PAD = _NW * _CHUNK


def _body(src_hbm, dst_hbm, edge_hbm, ts_hbm, idx_hbm,
          out_s, out_d, out_e, out_t,
          idx_v, buf0, buf1, sem0, sem1):
    wid = lax.axis_index("s") * _NC + lax.axis_index("c")
    base = wid * _CHUNK
    pltpu.sync_copy(idx_hbm.at[pl.ds(base, _CHUNK)], idx_v)

    h0 = pltpu.async_copy(src_hbm.at[idx_v], buf0, sem0)
    h1 = pltpu.async_copy(dst_hbm.at[idx_v], buf1, sem1)
    h0.wait()
    pltpu.sync_copy(buf0, out_s.at[pl.ds(base, _CHUNK)])
    h2 = pltpu.async_copy(edge_hbm.at[idx_v], buf0, sem0)
    h1.wait()
    pltpu.sync_copy(buf1, out_d.at[pl.ds(base, _CHUNK)])
    h3 = pltpu.async_copy(ts_hbm.at[idx_v], buf1, sem1)
    h2.wait()
    pltpu.sync_copy(buf0, out_e.at[pl.ds(base, _CHUNK)])
    h3.wait()
    pltpu.sync_copy(buf1, out_t.at[pl.ds(base, _CHUNK)])


def kernel(src, dst, edge_idxs, timestamps, idx):
    ts_i = lax.bitcast_convert_type(timestamps, jnp.int32)
    idx_p = jnp.concatenate([idx, jnp.zeros((_B_PAD - _B,), jnp.int32)])

    out_struct = jax.ShapeDtypeStruct((_B_PAD,), jnp.int32)
    call = pl.kernel(
        _body,
        out_type=(out_struct, out_struct, out_struct, out_struct),
        mesh=plsc.VectorSubcoreMesh(core_axis_name="c", subcore_axis_name="s"),
        scratch_types=[
            pltpu.VMEM((_CHUNK,), jnp.int32),
            pltpu.VMEM((_CHUNK,), jnp.int32),
            pltpu.VMEM((_CHUNK,), jnp.int32),
            pltpu.SemaphoreType.DMA,
            pltpu.SemaphoreType.DMA,
        ],
    )
    s, d, e, t = call(src, dst, edge_idxs, ts_i, idx_p)
    return (s[:_B], d[:_B], e[:_B],
            lax.bitcast_convert_type(t[:_B], jnp.float32))


# speedup vs baseline: 1.5205x; 1.5205x over previous
import jax
import jax.numpy as jnp
from jax import lax
from jax.experimental import pallas as pl
from jax.experimental.pallas import tpu as pltpu
from jax.experimental.pallas import tpu_sc as plsc

_N = 5_000_000
_B = 1_000_000

_NC = 2
_NS = 16
_NW = _NC * _NS
_CHUNK = 31_360


def _body(src_hbm, dst_hbm, edge_hbm, ts_hbm, idx_hbm,
          out_s, out_d, out_e, out_t,
          idx_v, buf0, buf1, buft, sem0, sem1, semt):
    wid = lax.axis_index("s") * _NC + lax.axis_index("c")
    base = lax.min(wid * _CHUNK, _B - _CHUNK)
    pltpu.sync_copy(idx_hbm.at[pl.ds(base, _CHUNK)], idx_v)

    h0 = pltpu.async_copy(src_hbm.at[idx_v], buf0, sem0)
    h1 = pltpu.async_copy(dst_hbm.at[idx_v], buf1, sem1)
    ht = pltpu.async_copy(ts_hbm.at[idx_v], buft, semt)
    h0.wait()
    pltpu.sync_copy(buf0, out_s.at[pl.ds(base, _CHUNK)])
    h2 = pltpu.async_copy(edge_hbm.at[idx_v], buf0, sem0)
    h1.wait()
    pltpu.sync_copy(buf1, out_d.at[pl.ds(base, _CHUNK)])
    h2.wait()
    pltpu.sync_copy(buf0, out_e.at[pl.ds(base, _CHUNK)])
    ht.wait()
    pltpu.sync_copy(buft, out_t.at[pl.ds(base, _CHUNK)])


def kernel(src, dst, edge_idxs, timestamps, idx):
    i32_out = jax.ShapeDtypeStruct((_B,), jnp.int32)
    f32_out = jax.ShapeDtypeStruct((_B,), jnp.float32)
    call = pl.kernel(
        _body,
        out_type=(i32_out, i32_out, i32_out, f32_out),
        mesh=plsc.VectorSubcoreMesh(core_axis_name="c", subcore_axis_name="s"),
        scratch_types=[
            pltpu.VMEM((_CHUNK,), jnp.int32),
            pltpu.VMEM((_CHUNK,), jnp.int32),
            pltpu.VMEM((_CHUNK,), jnp.int32),
            pltpu.VMEM((_CHUNK,), jnp.float32),
            pltpu.SemaphoreType.DMA,
            pltpu.SemaphoreType.DMA,
            pltpu.SemaphoreType.DMA,
        ],
    )
    return call(src, dst, edge_idxs, timestamps, idx)
